# Initial kernel scaffold; baseline (speedup 1.0000x reference)
import jax
import jax.numpy as jnp
from jax.experimental import pallas as pl


def kernel(coord, numbers, charge, afv_w, comb_a, comb_q, mlp1_params, mlp2_params):
    raise NotImplementedError("write your pallas kernel here")


# speedup vs baseline: 2.8963x; 2.8963x over previous
import math
import numpy as np
import jax
import jax.numpy as jnp
from jax.experimental import pallas as pl
from jax.experimental.pallas import tpu as pltpu

_NF = 128
_NS = 16
_NC = 8
_RC = 5.0
_RMIN = 0.8
_B = 128
_N = 64
_MB = 2
_ETA = 0.5 * (_NS / (_RC - _RMIN)) ** 2
_DELTA = (_RC - _RMIN) / (_NS - 1)


def _fused(coordn_ref, coordm_ref, numrc_ref, numbn_ref, charge_ref,
           afv_ref, comb_at_ref, comb_qt_ref,
           w1a_ref, w1s_ref, w1v_ref, b1_ref, w1b_ref, b1b_ref, w1c_ref, b1c_ref,
           w2a_ref, w2q_ref, w2s_ref, w2v_ref, w2qs_ref, w2qv_ref, b2_ref,
           w2b_ref, b2b_ref, w2c_ref, b2c_ref,
           aim_ref, qout_ref):
    MB, N, F, S, C = _MB, _N, _NF, _NS, _NC
    R = MB * N
    nums = numrc_ref[0]
    padr = nums == 0
    pad3 = padr.reshape(MB, N, 1)
    num_m = numbn_ref[0]
    valid_m = (num_m != 0).reshape(MB, 1, N)

    oh = (nums == jax.lax.broadcasted_iota(jnp.int32, (1, 64), 1)).astype(jnp.float32)
    a = oh @ afv_ref[...]

    cn = coordn_ref[...]
    cm = coordm_ref[...]
    rijs = [cn[:, dd] - cm[:, dd] for dd in range(3)]
    d2 = rijs[0] * rijs[0] + rijs[1] * rijs[1] + rijs[2] * rijs[2]
    d = jnp.sqrt(d2 + 1e-12)
    inv_d = 1.0 / d
    ums = [r * inv_d for r in rijs]
    ii = jax.lax.broadcasted_iota(jnp.int32, (N, N), 0)
    jj = jax.lax.broadcasted_iota(jnp.int32, (N, N), 1)
    eye = (ii == jj)[None]
    mask = (~eye) & (~pad3) & valid_m & (d < _RC)
    dc = jnp.minimum(d, _RC)
    fcm = jnp.where(mask, 0.5 * jnp.cos(np.pi / _RC * dc) + 0.5, 0.0)

    t = jnp.exp((2.0 * _ETA * _DELTA) * dc)
    gs_list = []
    g = None
    for s in range(S):
        if s % 4 == 0:
            sh_s = _RMIN + s * _DELTA
            g = jnp.exp(-_ETA * (dc - sh_s) ** 2) * fcm
        else:
            sh_prev = _RMIN + (s - 1) * _DELTA
            cnst = math.exp(-_ETA * (2.0 * _DELTA * sh_prev + _DELTA * _DELTA))
            g = g * t * cnst
        gs_list.append(g)
    gsm = jnp.stack(gs_list, axis=1)
    gs2 = gsm.reshape(MB, S * N, N)
    cdn = (((1,), (1,)), ((), ()))
    gca = jax.lax.dot_general(comb_at_ref[...], gsm, cdn).transpose(1, 0, 2, 3)
    gcq = jax.lax.dot_general(comb_qt_ref[...], gsm, cdn).transpose(1, 0, 2, 3)
    td_a = [(u[:, None] * gca).reshape(MB, C * N, N) for u in ums]
    td_q = [(u[:, None] * gcq).reshape(MB, C * N, N) for u in ums]
    dn = (((2,), (1,)), ((0,), (0,)))

    def conv_a(feat):
        f3 = feat.reshape(MB, N, F)
        s4 = jax.lax.dot_general(gs2, f3, dn).reshape(MB, S, N, F)
        acc = None
        for td in td_a:
            vv = jax.lax.dot_general(td, f3, dn)
            acc = vv * vv if acc is None else acc + vv * vv
        v4 = jnp.sqrt(acc + 1e-12).reshape(MB, C, N, F)
        return s4, v4

    def conv_head(feat, s4, v4, wa_ref, ws_ref, wv_ref, b_ref):
        h = feat @ wa_ref[...] + b_ref[...]
        for s in range(S):
            h = h + s4[:, s].reshape(R, F) @ ws_ref[s]
        for c in range(C):
            h = h + v4[:, c].reshape(R, F) @ wv_ref[c]
        return h

    def mlp_tail(h, wb, bb, wc, bc, last_linear):
        h = jax.nn.gelu(h)
        h = h @ wb + bb
        h = jax.nn.gelu(h)
        h = h @ wc + bc
        if not last_linear:
            h = jax.nn.gelu(h)
        return h

    s4, v4 = conv_a(a)
    h = conv_head(a, s4, v4, w1a_ref, w1s_ref, w1v_ref, b1_ref)
    out1 = mlp_tail(h, w1b_ref[...], b1b_ref[...], w1c_ref[...], b1c_ref[...],
                    last_linear=True)
    out1 = jnp.where(padr, 0.0, out1)
    _a = out1[:, :F]
    q3 = out1[:, F:F + 1].reshape(MB, N, 1)
    f3c = out1[:, F + 1:F + 2].reshape(MB, N, 1)

    vmask = (~pad3).astype(jnp.float32)
    f_m = jnp.where(pad3, -1e9, f3c)
    w = jnp.exp(f_m - jnp.max(f_m, axis=1, keepdims=True)) * vmask
    w = w / (jnp.sum(w, axis=1, keepdims=True) + 1e-12)
    charge_col = charge_ref[0].reshape(MB, 1, 1)
    tot = jnp.sum(q3 * vmask, axis=1, keepdims=True)
    q3 = (q3 + (charge_col - tot) * w) * vmask

    a2 = a + _a
    s4b, v4b = conv_a(a2)
    sq4 = jax.lax.dot_general(gs2, q3, dn).reshape(MB, S, N, 1)
    accq = None
    for td in td_q:
        tq = jax.lax.dot_general(td, q3, dn)
        accq = tq * tq if accq is None else accq + tq * tq
    vq4 = jnp.sqrt(accq + 1e-12).reshape(MB, C, N, 1)
    sq = jnp.concatenate([sq4[:, s].reshape(R, 1) for s in range(S)], axis=-1)
    vq = jnp.concatenate([vq4[:, c].reshape(R, 1) for c in range(C)], axis=-1)
    h2 = conv_head(a2, s4b, v4b, w2a_ref, w2s_ref, w2v_ref, b2_ref)
    h2 = h2 + q3.reshape(R, 1) * w2q_ref[...] + sq @ w2qs_ref[...] + vq @ w2qv_ref[...]
    aim = mlp_tail(h2, w2b_ref[...], b2b_ref[...], w2c_ref[...], b2c_ref[...],
                   last_linear=False)
    aim_ref[...] = aim.reshape(MB, N, 256)
    qout_ref[...] = q3


def kernel(coord, numbers, charge, afv_w, comb_a, comb_q, mlp1_params, mlp2_params):
    F, S, C = _NF, _NS, _NC
    (W1, b1), (W1b, b1b), (W1c, b1c) = mlp1_params
    (W2, b2), (W2b, b2b), (W2c, b2c) = mlp2_params
    H = 256

    w1s = W1[F:F + S * F].reshape(F, S, H).transpose(1, 0, 2)
    w1v = W1[F + S * F:].reshape(F, C, H).transpose(1, 0, 2)
    w1a = W1[:F]
    colp = np.concatenate([np.arange(2, F + 2), np.array([0, 1])])
    w1c = W1c[:, colp]
    b1c_p = b1c[colp]
    w2a = W2[:F]
    w2q = W2[F:F + 1]
    w2s = W2[F + 1:F + 1 + S * F].reshape(F, S, H).transpose(1, 0, 2)
    w2v = W2[F + 1 + S * F:F + 1 + (S + C) * F].reshape(F, C, H).transpose(1, 0, 2)
    w2qs = W2[F + 1 + (S + C) * F:F + 1 + (S + C) * F + S]
    w2qv = W2[F + 1 + (S + C) * F + S:]

    coordn = coord.transpose(0, 2, 1).reshape(_B, 3, _N, 1)
    coordm = coord.transpose(0, 2, 1).reshape(_B, 3, 1, _N)
    numrc = numbers.reshape(_B // _MB, _MB * _N, 1)
    numbn = numbers.reshape(_B // _MB, _MB, _N)
    charge3 = charge.reshape(_B // _MB, _MB, 1)

    def blk3(i):
        return (i, 0, 0)

    def blk4(i):
        return (i, 0, 0, 0)

    def full(shape):
        nd = len(shape)
        return pl.BlockSpec(shape, lambda i, _n=nd: (0,) * _n)

    in_specs = [
        pl.BlockSpec((_MB, 3, _N, 1), blk4),
        pl.BlockSpec((_MB, 3, 1, _N), blk4),
        pl.BlockSpec((1, _MB * _N, 1), blk3),
        pl.BlockSpec((1, _MB, _N), blk3),
        pl.BlockSpec((1, _MB, 1), blk3),
        full((64, F)), full((C, S)), full((C, S)),
        full((F, H)), full((S, F, H)), full((C, F, H)), full((1, H)),
        full((H, H)), full((1, H)), full((H, F + 2)), full((1, F + 2)),
        full((F, H)), full((1, H)), full((S, F, H)), full((C, F, H)),
        full((S, H)), full((C, H)), full((1, H)),
        full((H, H)), full((1, H)), full((H, H)), full((1, H)),
    ]
    out_specs = [
        pl.BlockSpec((_MB, _N, 256), blk3),
        pl.BlockSpec((_MB, _N, 1), blk3),
    ]
    aim, q3 = pl.pallas_call(
        _fused,
        grid=(_B // _MB,),
        in_specs=in_specs,
        out_specs=out_specs,
        out_shape=[jax.ShapeDtypeStruct((_B, _N, 256), jnp.float32),
                   jax.ShapeDtypeStruct((_B, _N, 1), jnp.float32)],
        compiler_params=pltpu.CompilerParams(dimension_semantics=("parallel",)),
    )(coordn, coordm, numrc, numbn, charge3, afv_w,
      comb_a.transpose(1, 0), comb_q.transpose(1, 0),
      w1a, w1s, w1v, b1.reshape(1, -1), W1b, b1b.reshape(1, -1),
      w1c, b1c_p.reshape(1, -1),
      w2a, w2q, w2s, w2v, w2qs, w2qv, b2.reshape(1, -1),
      W2b, b2b.reshape(1, -1), W2c, b2c.reshape(1, -1))
    return aim, q3[..., 0]


# speedup vs baseline: 3.5410x; 1.2226x over previous
import math
import numpy as np
import jax
import jax.numpy as jnp
from jax.experimental import pallas as pl
from jax.experimental.pallas import tpu as pltpu

_NF = 128
_NS = 16
_NC = 8
_RC = 5.0
_RMIN = 0.8
_B = 128
_N = 64
_MB = 4
_ETA = 0.5 * (_NS / (_RC - _RMIN)) ** 2
_DELTA = (_RC - _RMIN) / (_NS - 1)


def _fused(coordn_ref, coordm_ref, numrc_ref, numbn_ref, charge_ref,
           afv_ref, comb_at_ref, comb_qt_ref,
           w1a_ref, w1s_ref, w1v_ref, b1_ref, w1b_ref, b1b_ref, w1c_ref, b1c_ref,
           w2a_ref, w2q_ref, w2s_ref, w2v_ref, w2qs_ref, w2qv_ref, b2_ref,
           w2b_ref, b2b_ref, w2c_ref, b2c_ref,
           aim_ref, qout_ref):
    MB, N, F, S, C = _MB, _N, _NF, _NS, _NC
    R = MB * N
    nums = numrc_ref[0]
    padr = nums == 0
    pad3 = padr.reshape(MB, N, 1)
    num_m = numbn_ref[0]
    valid_m = (num_m != 0).reshape(MB, 1, N)

    oh = (nums == jax.lax.broadcasted_iota(jnp.int32, (1, 64), 1)).astype(jnp.float32)
    a = oh @ afv_ref[...]

    cn = coordn_ref[...]
    cm = coordm_ref[...]
    rijs = [cn[:, dd] - cm[:, dd] for dd in range(3)]
    d2 = rijs[0] * rijs[0] + rijs[1] * rijs[1] + rijs[2] * rijs[2]
    d = jnp.sqrt(d2 + 1e-12)
    inv_d = 1.0 / d
    ums = [r * inv_d for r in rijs]
    ii = jax.lax.broadcasted_iota(jnp.int32, (N, N), 0)
    jj = jax.lax.broadcasted_iota(jnp.int32, (N, N), 1)
    eye = (ii == jj)[None]
    mask = (~eye) & (~pad3) & valid_m & (d < _RC)
    dc = jnp.minimum(d, _RC)
    fcm = jnp.where(mask, 0.5 * jnp.cos(np.pi / _RC * dc) + 0.5, 0.0)

    t = jnp.exp((2.0 * _ETA * _DELTA) * dc)
    gs_list = []
    g = None
    for s in range(S):
        if s % 4 == 0:
            sh_s = _RMIN + s * _DELTA
            g = jnp.exp(-_ETA * (dc - sh_s) ** 2) * fcm
        else:
            sh_prev = _RMIN + (s - 1) * _DELTA
            cnst = math.exp(-_ETA * (2.0 * _DELTA * sh_prev + _DELTA * _DELTA))
            g = g * t * cnst
        gs_list.append(g)
    gsm = jnp.stack(gs_list, axis=1)
    gs2 = gsm.reshape(MB, S * N, N)
    cdn = (((1,), (1,)), ((), ()))
    gca = jax.lax.dot_general(comb_at_ref[...], gsm, cdn).transpose(1, 0, 2, 3)
    gcq = jax.lax.dot_general(comb_qt_ref[...], gsm, cdn).transpose(1, 0, 2, 3)
    td_a = [(u[:, None] * gca).reshape(MB, C * N, N) for u in ums]
    td_q = [(u[:, None] * gcq).reshape(MB, C * N, N) for u in ums]
    dn = (((2,), (1,)), ((0,), (0,)))

    def conv_a(feat):
        f3 = feat.reshape(MB, N, F)
        s4 = jax.lax.dot_general(gs2, f3, dn).reshape(MB, S, N, F)
        acc = None
        for td in td_a:
            vv = jax.lax.dot_general(td, f3, dn)
            acc = vv * vv if acc is None else acc + vv * vv
        v4 = jnp.sqrt(acc + 1e-12).reshape(MB, C, N, F)
        return s4, v4

    def conv_head(feat, s4, v4, wa_ref, ws_ref, wv_ref, b_ref):
        h = feat @ wa_ref[...] + b_ref[...]
        for s in range(S):
            h = h + s4[:, s].reshape(R, F) @ ws_ref[s]
        for c in range(C):
            h = h + v4[:, c].reshape(R, F) @ wv_ref[c]
        return h

    def mlp_tail(h, wb, bb, wc, bc, last_linear):
        h = jax.nn.gelu(h)
        h = h @ wb + bb
        h = jax.nn.gelu(h)
        h = h @ wc + bc
        if not last_linear:
            h = jax.nn.gelu(h)
        return h

    s4, v4 = conv_a(a)
    h = conv_head(a, s4, v4, w1a_ref, w1s_ref, w1v_ref, b1_ref)
    out1 = mlp_tail(h, w1b_ref[...], b1b_ref[...], w1c_ref[...], b1c_ref[...],
                    last_linear=True)
    out1 = jnp.where(padr, 0.0, out1)
    _a = out1[:, :F]
    q3 = out1[:, F:F + 1].reshape(MB, N, 1)
    f3c = out1[:, F + 1:F + 2].reshape(MB, N, 1)

    vmask = (~pad3).astype(jnp.float32)
    f_m = jnp.where(pad3, -1e9, f3c)
    w = jnp.exp(f_m - jnp.max(f_m, axis=1, keepdims=True)) * vmask
    w = w / (jnp.sum(w, axis=1, keepdims=True) + 1e-12)
    charge_col = charge_ref[0].reshape(MB, 1, 1)
    tot = jnp.sum(q3 * vmask, axis=1, keepdims=True)
    q3 = (q3 + (charge_col - tot) * w) * vmask

    a2 = a + _a
    s4b, v4b = conv_a(a2)
    sq4 = jax.lax.dot_general(gs2, q3, dn).reshape(MB, S, N, 1)
    accq = None
    for td in td_q:
        tq = jax.lax.dot_general(td, q3, dn)
        accq = tq * tq if accq is None else accq + tq * tq
    vq4 = jnp.sqrt(accq + 1e-12).reshape(MB, C, N, 1)
    sq = jnp.concatenate([sq4[:, s].reshape(R, 1) for s in range(S)], axis=-1)
    vq = jnp.concatenate([vq4[:, c].reshape(R, 1) for c in range(C)], axis=-1)
    h2 = conv_head(a2, s4b, v4b, w2a_ref, w2s_ref, w2v_ref, b2_ref)
    h2 = h2 + q3.reshape(R, 1) * w2q_ref[...] + sq @ w2qs_ref[...] + vq @ w2qv_ref[...]
    aim = mlp_tail(h2, w2b_ref[...], b2b_ref[...], w2c_ref[...], b2c_ref[...],
                   last_linear=False)
    aim_ref[...] = aim.reshape(MB, N, 256)
    qout_ref[...] = q3


def kernel(coord, numbers, charge, afv_w, comb_a, comb_q, mlp1_params, mlp2_params):
    F, S, C = _NF, _NS, _NC
    (W1, b1), (W1b, b1b), (W1c, b1c) = mlp1_params
    (W2, b2), (W2b, b2b), (W2c, b2c) = mlp2_params
    H = 256

    w1s = W1[F:F + S * F].reshape(F, S, H).transpose(1, 0, 2)
    w1v = W1[F + S * F:].reshape(F, C, H).transpose(1, 0, 2)
    w1a = W1[:F]
    colp = np.concatenate([np.arange(2, F + 2), np.array([0, 1])])
    w1c = W1c[:, colp]
    b1c_p = b1c[colp]
    w2a = W2[:F]
    w2q = W2[F:F + 1]
    w2s = W2[F + 1:F + 1 + S * F].reshape(F, S, H).transpose(1, 0, 2)
    w2v = W2[F + 1 + S * F:F + 1 + (S + C) * F].reshape(F, C, H).transpose(1, 0, 2)
    w2qs = W2[F + 1 + (S + C) * F:F + 1 + (S + C) * F + S]
    w2qv = W2[F + 1 + (S + C) * F + S:]

    coordn = coord.transpose(0, 2, 1).reshape(_B, 3, _N, 1)
    coordm = coord.transpose(0, 2, 1).reshape(_B, 3, 1, _N)
    numrc = numbers.reshape(_B // _MB, _MB * _N, 1)
    numbn = numbers.reshape(_B // _MB, _MB, _N)
    charge3 = charge.reshape(_B // _MB, _MB, 1)

    def blk3(i):
        return (i, 0, 0)

    def blk4(i):
        return (i, 0, 0, 0)

    def full(shape):
        nd = len(shape)
        return pl.BlockSpec(shape, lambda i, _n=nd: (0,) * _n)

    in_specs = [
        pl.BlockSpec((_MB, 3, _N, 1), blk4),
        pl.BlockSpec((_MB, 3, 1, _N), blk4),
        pl.BlockSpec((1, _MB * _N, 1), blk3),
        pl.BlockSpec((1, _MB, _N), blk3),
        pl.BlockSpec((1, _MB, 1), blk3),
        full((64, F)), full((C, S)), full((C, S)),
        full((F, H)), full((S, F, H)), full((C, F, H)), full((1, H)),
        full((H, H)), full((1, H)), full((H, F + 2)), full((1, F + 2)),
        full((F, H)), full((1, H)), full((S, F, H)), full((C, F, H)),
        full((S, H)), full((C, H)), full((1, H)),
        full((H, H)), full((1, H)), full((H, H)), full((1, H)),
    ]
    out_specs = [
        pl.BlockSpec((_MB, _N, 256), blk3),
        pl.BlockSpec((_MB, _N, 1), blk3),
    ]
    aim, q3 = pl.pallas_call(
        _fused,
        grid=(_B // _MB,),
        in_specs=in_specs,
        out_specs=out_specs,
        out_shape=[jax.ShapeDtypeStruct((_B, _N, 256), jnp.float32),
                   jax.ShapeDtypeStruct((_B, _N, 1), jnp.float32)],
        compiler_params=pltpu.CompilerParams(dimension_semantics=("parallel",)),
    )(coordn, coordm, numrc, numbn, charge3, afv_w,
      comb_a.transpose(1, 0), comb_q.transpose(1, 0),
      w1a, w1s, w1v, b1.reshape(1, -1), W1b, b1b.reshape(1, -1),
      w1c, b1c_p.reshape(1, -1),
      w2a, w2q, w2s, w2v, w2qs, w2qv, b2.reshape(1, -1),
      W2b, b2b.reshape(1, -1), W2c, b2c.reshape(1, -1))
    return aim, q3[..., 0]


# speedup vs baseline: 3.5434x; 1.0007x over previous
import math
import numpy as np
import jax
import jax.numpy as jnp
from jax.experimental import pallas as pl
from jax.experimental.pallas import tpu as pltpu

_NF = 128
_NS = 16
_NC = 8
_RC = 5.0
_RMIN = 0.8
_B = 128
_N = 64
_MB = 8
_ETA = 0.5 * (_NS / (_RC - _RMIN)) ** 2
_DELTA = (_RC - _RMIN) / (_NS - 1)


def _fused(coordn_ref, coordm_ref, numrc_ref, numbn_ref, charge_ref,
           afv_ref, comb_at_ref, comb_qt_ref,
           w1a_ref, w1s_ref, w1v_ref, b1_ref, w1b_ref, b1b_ref, w1c_ref, b1c_ref,
           w2a_ref, w2q_ref, w2s_ref, w2v_ref, w2qs_ref, w2qv_ref, b2_ref,
           w2b_ref, b2b_ref, w2c_ref, b2c_ref,
           aim_ref, qout_ref):
    MB, N, F, S, C = _MB, _N, _NF, _NS, _NC
    R = MB * N
    nums = numrc_ref[0]
    padr = nums == 0
    pad3 = padr.reshape(MB, N, 1)
    num_m = numbn_ref[0]
    valid_m = (num_m != 0).reshape(MB, 1, N)

    oh = (nums == jax.lax.broadcasted_iota(jnp.int32, (1, 64), 1)).astype(jnp.float32)
    a = oh @ afv_ref[...]

    cn = coordn_ref[...]
    cm = coordm_ref[...]
    rijs = [cn[:, dd] - cm[:, dd] for dd in range(3)]
    d2 = rijs[0] * rijs[0] + rijs[1] * rijs[1] + rijs[2] * rijs[2]
    d = jnp.sqrt(d2 + 1e-12)
    inv_d = 1.0 / d
    ums = [r * inv_d for r in rijs]
    ii = jax.lax.broadcasted_iota(jnp.int32, (N, N), 0)
    jj = jax.lax.broadcasted_iota(jnp.int32, (N, N), 1)
    eye = (ii == jj)[None]
    mask = (~eye) & (~pad3) & valid_m & (d < _RC)
    dc = jnp.minimum(d, _RC)
    fcm = jnp.where(mask, 0.5 * jnp.cos(np.pi / _RC * dc) + 0.5, 0.0)

    t = jnp.exp((2.0 * _ETA * _DELTA) * dc)
    gs_list = []
    g = None
    for s in range(S):
        if s % 4 == 0:
            sh_s = _RMIN + s * _DELTA
            g = jnp.exp(-_ETA * (dc - sh_s) ** 2) * fcm
        else:
            sh_prev = _RMIN + (s - 1) * _DELTA
            cnst = math.exp(-_ETA * (2.0 * _DELTA * sh_prev + _DELTA * _DELTA))
            g = g * t * cnst
        gs_list.append(g)
    gsm = jnp.stack(gs_list, axis=1)
    gs2 = gsm.reshape(MB, S * N, N)
    cdn = (((1,), (1,)), ((), ()))
    gca = jax.lax.dot_general(comb_at_ref[...], gsm, cdn).transpose(1, 0, 2, 3)
    gcq = jax.lax.dot_general(comb_qt_ref[...], gsm, cdn).transpose(1, 0, 2, 3)
    td_a = [(u[:, None] * gca).reshape(MB, C * N, N) for u in ums]
    td_q = [(u[:, None] * gcq).reshape(MB, C * N, N) for u in ums]
    dn = (((2,), (1,)), ((0,), (0,)))

    def conv_a(feat):
        f3 = feat.reshape(MB, N, F)
        s4 = jax.lax.dot_general(gs2, f3, dn).reshape(MB, S, N, F)
        acc = None
        for td in td_a:
            vv = jax.lax.dot_general(td, f3, dn)
            acc = vv * vv if acc is None else acc + vv * vv
        v4 = jnp.sqrt(acc + 1e-12).reshape(MB, C, N, F)
        return s4, v4

    def conv_head(feat, s4, v4, wa_ref, ws_ref, wv_ref, b_ref):
        h = feat @ wa_ref[...] + b_ref[...]
        for s in range(S):
            h = h + s4[:, s].reshape(R, F) @ ws_ref[s]
        for c in range(C):
            h = h + v4[:, c].reshape(R, F) @ wv_ref[c]
        return h

    def mlp_tail(h, wb, bb, wc, bc, last_linear):
        h = jax.nn.gelu(h)
        h = h @ wb + bb
        h = jax.nn.gelu(h)
        h = h @ wc + bc
        if not last_linear:
            h = jax.nn.gelu(h)
        return h

    s4, v4 = conv_a(a)
    h = conv_head(a, s4, v4, w1a_ref, w1s_ref, w1v_ref, b1_ref)
    out1 = mlp_tail(h, w1b_ref[...], b1b_ref[...], w1c_ref[...], b1c_ref[...],
                    last_linear=True)
    out1 = jnp.where(padr, 0.0, out1)
    _a = out1[:, :F]
    q3 = out1[:, F:F + 1].reshape(MB, N, 1)
    f3c = out1[:, F + 1:F + 2].reshape(MB, N, 1)

    vmask = (~pad3).astype(jnp.float32)
    f_m = jnp.where(pad3, -1e9, f3c)
    w = jnp.exp(f_m - jnp.max(f_m, axis=1, keepdims=True)) * vmask
    w = w / (jnp.sum(w, axis=1, keepdims=True) + 1e-12)
    charge_col = charge_ref[0].reshape(MB, 1, 1)
    tot = jnp.sum(q3 * vmask, axis=1, keepdims=True)
    q3 = (q3 + (charge_col - tot) * w) * vmask

    a2 = a + _a
    s4b, v4b = conv_a(a2)
    sq4 = jax.lax.dot_general(gs2, q3, dn).reshape(MB, S, N, 1)
    accq = None
    for td in td_q:
        tq = jax.lax.dot_general(td, q3, dn)
        accq = tq * tq if accq is None else accq + tq * tq
    vq4 = jnp.sqrt(accq + 1e-12).reshape(MB, C, N, 1)
    sq = jnp.concatenate([sq4[:, s].reshape(R, 1) for s in range(S)], axis=-1)
    vq = jnp.concatenate([vq4[:, c].reshape(R, 1) for c in range(C)], axis=-1)
    h2 = conv_head(a2, s4b, v4b, w2a_ref, w2s_ref, w2v_ref, b2_ref)
    h2 = h2 + q3.reshape(R, 1) * w2q_ref[...] + sq @ w2qs_ref[...] + vq @ w2qv_ref[...]
    aim = mlp_tail(h2, w2b_ref[...], b2b_ref[...], w2c_ref[...], b2c_ref[...],
                   last_linear=False)
    aim_ref[...] = aim.reshape(MB, N, 256)
    qout_ref[...] = q3


def kernel(coord, numbers, charge, afv_w, comb_a, comb_q, mlp1_params, mlp2_params):
    F, S, C = _NF, _NS, _NC
    (W1, b1), (W1b, b1b), (W1c, b1c) = mlp1_params
    (W2, b2), (W2b, b2b), (W2c, b2c) = mlp2_params
    H = 256

    w1s = W1[F:F + S * F].reshape(F, S, H).transpose(1, 0, 2)
    w1v = W1[F + S * F:].reshape(F, C, H).transpose(1, 0, 2)
    w1a = W1[:F]
    colp = np.concatenate([np.arange(2, F + 2), np.array([0, 1])])
    w1c = W1c[:, colp]
    b1c_p = b1c[colp]
    w2a = W2[:F]
    w2q = W2[F:F + 1]
    w2s = W2[F + 1:F + 1 + S * F].reshape(F, S, H).transpose(1, 0, 2)
    w2v = W2[F + 1 + S * F:F + 1 + (S + C) * F].reshape(F, C, H).transpose(1, 0, 2)
    w2qs = W2[F + 1 + (S + C) * F:F + 1 + (S + C) * F + S]
    w2qv = W2[F + 1 + (S + C) * F + S:]

    coordn = coord.transpose(0, 2, 1).reshape(_B, 3, _N, 1)
    coordm = coord.transpose(0, 2, 1).reshape(_B, 3, 1, _N)
    numrc = numbers.reshape(_B // _MB, _MB * _N, 1)
    numbn = numbers.reshape(_B // _MB, _MB, _N)
    charge3 = charge.reshape(_B // _MB, _MB, 1)

    def blk3(i):
        return (i, 0, 0)

    def blk4(i):
        return (i, 0, 0, 0)

    def full(shape):
        nd = len(shape)
        return pl.BlockSpec(shape, lambda i, _n=nd: (0,) * _n)

    in_specs = [
        pl.BlockSpec((_MB, 3, _N, 1), blk4),
        pl.BlockSpec((_MB, 3, 1, _N), blk4),
        pl.BlockSpec((1, _MB * _N, 1), blk3),
        pl.BlockSpec((1, _MB, _N), blk3),
        pl.BlockSpec((1, _MB, 1), blk3),
        full((64, F)), full((C, S)), full((C, S)),
        full((F, H)), full((S, F, H)), full((C, F, H)), full((1, H)),
        full((H, H)), full((1, H)), full((H, F + 2)), full((1, F + 2)),
        full((F, H)), full((1, H)), full((S, F, H)), full((C, F, H)),
        full((S, H)), full((C, H)), full((1, H)),
        full((H, H)), full((1, H)), full((H, H)), full((1, H)),
    ]
    out_specs = [
        pl.BlockSpec((_MB, _N, 256), blk3),
        pl.BlockSpec((_MB, _N, 1), blk3),
    ]
    aim, q3 = pl.pallas_call(
        _fused,
        grid=(_B // _MB,),
        in_specs=in_specs,
        out_specs=out_specs,
        out_shape=[jax.ShapeDtypeStruct((_B, _N, 256), jnp.float32),
                   jax.ShapeDtypeStruct((_B, _N, 1), jnp.float32)],
        compiler_params=pltpu.CompilerParams(dimension_semantics=("parallel",)),
    )(coordn, coordm, numrc, numbn, charge3, afv_w,
      comb_a.transpose(1, 0), comb_q.transpose(1, 0),
      w1a, w1s, w1v, b1.reshape(1, -1), W1b, b1b.reshape(1, -1),
      w1c, b1c_p.reshape(1, -1),
      w2a, w2q, w2s, w2v, w2qs, w2qv, b2.reshape(1, -1),
      W2b, b2b.reshape(1, -1), W2c, b2c.reshape(1, -1))
    return aim, q3[..., 0]
